# Initial kernel scaffold; baseline (speedup 1.0000x reference)
import jax
import jax.numpy as jnp
from jax.experimental import pallas as pl


def kernel(pc, query, params):
    raise NotImplementedError("write your pallas kernel here")


# speedup vs baseline: 6.3306x; 6.3306x over previous
import functools

import jax
import jax.numpy as jnp
from jax import lax
from jax.experimental import pallas as pl
from jax.experimental.pallas import tpu as pltpu
from jax.experimental.pallas import tpu_sc as plsc

EPS = 1e-5
F32 = jnp.float32


def _dotf(a, b):
    return jnp.dot(a, b, preferred_element_type=F32)


def _dot_c1c1(a, b, precision=None):
    return lax.dot_general(a, b, (((1,), (1,)), ((), ())),
                           preferred_element_type=F32, precision=precision)


def _fps_body(S, N, x_ref, cent_ref):
    x0 = x_ref[0]
    x1 = x_ref[1]
    x2 = x_ref[2]
    B = x0.shape[0]
    iota_n = lax.broadcasted_iota(jnp.int32, (B, N), 1)
    iota_s = lax.broadcasted_iota(jnp.int32, (B, S), 1)

    def body(i, st):
        cent, dist, far = st
        cent = jnp.where(iota_s == i, far, cent)
        ohf = (iota_n == far).astype(F32)
        c0 = jnp.sum(x0 * ohf, axis=1, keepdims=True)
        c1 = jnp.sum(x1 * ohf, axis=1, keepdims=True)
        c2 = jnp.sum(x2 * ohf, axis=1, keepdims=True)
        d = (x0 - c0) ** 2 + (x1 - c1) ** 2 + (x2 - c2) ** 2
        dist = jnp.minimum(dist, d)
        m = jnp.max(dist, axis=1, keepdims=True)
        far = jnp.min(jnp.where(dist == m, iota_n, N), axis=1, keepdims=True)
        return cent, dist, far

    zn = jnp.where(x0 > 1e30, 1, 0)
    cent0 = zn[:, :S] if S <= N else jnp.concatenate(
        [zn] * (S // N) + [zn[:, :S % N]], axis=1)
    dist0 = jnp.where(x0 > 1e30, 0.0, 1e10)
    far0 = zn[:, 0:1]
    cent, _, _ = lax.fori_loop(0, S, body, (cent0, dist0, far0))
    cent_ref[...] = cent


def _fps(xyz_3bn, S):
    _, B, N = xyz_3bn.shape
    return pl.pallas_call(
        functools.partial(_fps_body, S, N),
        out_shape=jax.ShapeDtypeStruct((B, S), jnp.int32),
    )(xyz_3bn)


def _geo_body(N, S, K, bw, x_ref, fps_ref, invd_ref, nxyz_ref, knn_ref):
    X = x_ref[0]
    n2c = jnp.sum(X * X, axis=1, keepdims=True)
    ones13 = jnp.ones((1, 3), F32)
    n2r = _dot_c1c1(ones13, X * X, lax.Precision.HIGHEST)
    G = _dot_c1c1(X, X)
    Dfull = -2.0 * G + n2c + n2r
    gm = jnp.exp(Dfull * (-1.0 / (2.0 * bw * bw))) * (1.0 / (2.5 * bw))
    density = jnp.mean(gm, axis=1, keepdims=True)
    invd_ref[0] = 1.0 / density

    fidx = fps_ref[0]
    iota_sn = lax.broadcasted_iota(jnp.int32, (S, N), 1)
    oh = (iota_sn == fidx).astype(F32)
    Q = lax.dot_general(oh, X, (((1,), (0,)), ((), ())),
                        preferred_element_type=F32,
                        precision=lax.Precision.HIGHEST)
    nxyz_ref[0] = Q

    qn2 = jnp.sum(Q * Q, axis=1, keepdims=True)
    Dq = -2.0 * _dot_c1c1(Q, X) + qn2 + n2r
    for k in range(K):
        m = jnp.min(Dq, axis=1, keepdims=True)
        idx = jnp.min(jnp.where(Dq == m, iota_sn, N), axis=1, keepdims=True)
        knn_ref[0, :, k:k + 1] = idx
        Dq = jnp.where(iota_sn == idx, 1e30, Dq)


def _geo(xyz_bn3, fps_bs1, K, bw):
    B, N, _ = xyz_bn3.shape
    S = fps_bs1.shape[1]
    body = functools.partial(_geo_body, N, S, K, bw)
    invd, nxyz, knn = pl.pallas_call(
        body,
        grid=(B,),
        in_specs=[
            pl.BlockSpec((1, N, 3), lambda b: (b, 0, 0)),
            pl.BlockSpec((1, S, 1), lambda b: (b, 0, 0)),
        ],
        out_specs=[
            pl.BlockSpec((1, N, 1), lambda b: (b, 0, 0)),
            pl.BlockSpec((1, S, 3), lambda b: (b, 0, 0)),
            pl.BlockSpec((1, S, K), lambda b: (b, 0, 0)),
        ],
        out_shape=[
            jax.ShapeDtypeStruct((B, N, 1), F32),
            jax.ShapeDtypeStruct((B, S, 3), F32),
            jax.ShapeDtypeStruct((B, S, K), jnp.int32),
        ],
    )(xyz_bn3, fps_bs1)
    return invd, nxyz, knn


def _dens_body(N, bw, x_ref, invd_ref):
    X = x_ref[0]
    n2c = jnp.sum(X * X, axis=1, keepdims=True)
    ones13 = jnp.ones((1, 3), F32)
    n2r = _dot_c1c1(ones13, X * X, lax.Precision.HIGHEST)
    G = _dot_c1c1(X, X)
    Dfull = -2.0 * G + n2c + n2r
    gm = jnp.exp(Dfull * (-1.0 / (2.0 * bw * bw))) * (1.0 / (2.5 * bw))
    density = jnp.mean(gm, axis=1, keepdims=True)
    invd_ref[0] = 1.0 / density


def _dens(xyz_bn3, bw):
    B, N, _ = xyz_bn3.shape
    return pl.pallas_call(
        functools.partial(_dens_body, N, bw),
        grid=(B,),
        in_specs=[pl.BlockSpec((1, N, 3), lambda b: (b, 0, 0))],
        out_specs=pl.BlockSpec((1, N, 1), lambda b: (b, 0, 0)),
        out_shape=jax.ShapeDtypeStruct((B, N, 1), F32),
    )(xyz_bn3)


def _sc_gather(table, idx):
    V, D = table.shape
    Bi = idx.shape[0]
    info = plsc.get_sparse_core_info()
    NW = info.num_cores * info.num_subcores
    b_per_w = Bi // NW
    mesh = plsc.VectorSubcoreMesh(core_axis_name="c", subcore_axis_name="s")

    @functools.partial(
        pl.kernel, mesh=mesh,
        compiler_params=pltpu.CompilerParams(use_tc_tiling_on_sc=False),
        out_type=jax.ShapeDtypeStruct((Bi, D), F32),
        scratch_types=[
            pltpu.VMEM((b_per_w,), jnp.int32),
            pltpu.VMEM((b_per_w, D), F32),
            pltpu.SemaphoreType.DMA,
        ],
    )
    def k(table_hbm, idx_hbm, out_hbm, idx_v, rows_v, sem):
        wid = lax.axis_index("s") * info.num_cores + lax.axis_index("c")
        base = wid * b_per_w
        pltpu.sync_copy(idx_hbm.at[pl.ds(base, b_per_w)], idx_v)
        pltpu.async_copy(table_hbm.at[idx_v], rows_v, sem).wait()
        pltpu.sync_copy(rows_v, out_hbm.at[pl.ds(base, b_per_w)])

    return k(table, idx)


def _conv_bn_relu(x, chain_refs):
    for (wt_ref, b_ref, g_ref, bt_ref) in chain_refs:
        wt = wt_ref[...]
        if wt.shape[0] == 1:
            y = x * wt + b_ref[...]
        else:
            y = _dotf(x, wt) + b_ref[...]
        m = jnp.mean(y, axis=0, keepdims=True)
        v = jnp.mean((y - m) ** 2, axis=0, keepdims=True)
        y = (y - m) / jnp.sqrt(v + EPS) * g_ref[...] + bt_ref[...]
        x = jnp.maximum(y, 0.0)
    return x


def _make_grouped_body(Gp, K, P, C, center):
    R = Gp * K

    def body(*refs):
        i = 0
        g_ref = refs[i]; i += 1
        nx_ref = None
        if center:
            nx_ref = refs[i]; i += 1

        def take_chain(n):
            nonlocal i
            ch = []
            for _ in range(n):
                ch.append(tuple(refs[i:i + 4]))
                i += 4
            return ch

        mlp = take_chain(1)
        wn = take_chain(3)
        dn = take_chain(3)
        lot_ref, linb_ref, bg_ref, bb_ref = refs[i:i + 4]; i += 4
        out_ref = refs[i]

        g3 = g_ref[...]
        xyzn3 = g3[:, :, 0:3]
        if center:
            xyzn3 = xyzn3 - nx_ref[...]
        np3 = jnp.concatenate([xyzn3, g3[:, :, 3:3 + P]], axis=2)
        np2 = np3.reshape(R, 3 + P)
        xyzn2 = xyzn3.reshape(R, 3)
        invd3 = g3[:, :, 3 + P:4 + P]
        mx = jnp.max(invd3, axis=1, keepdims=True)
        ds2 = (invd3 / mx).reshape(R, 1)

        x = _conv_bn_relu(np2, mlp)
        d = _conv_bn_relu(ds2, dn)
        x = x * d
        w = _conv_bn_relu(xyzn2, wn)

        x3 = x.astype(jnp.bfloat16).astype(F32).reshape(Gp, K, C)
        w3 = w.astype(jnp.bfloat16).astype(F32).reshape(Gp, K, 16)
        acc = jnp.zeros((Gp, C), F32)
        for o in range(16):
            t = jnp.sum(x3 * w3[:, :, o:o + 1], axis=1)
            acc = acc + _dotf(t, lot_ref[o])
        y = acc + linb_ref[...]
        m = jnp.mean(y, axis=0, keepdims=True)
        v = jnp.mean((y - m) ** 2, axis=0, keepdims=True)
        y = (y - m) / jnp.sqrt(v + EPS) * bg_ref[...] + bb_ref[...]
        out_ref[...] = jnp.maximum(y, 0.0)

    return body


def _r2(a):
    return a.reshape(1, -1)


def _chain_args(chain):
    out = []
    for L in chain:
        out += [L['w'].T, _r2(L['b']), _r2(L['g']), _r2(L['beta'])]
    return out


def _grouped(g3, nx, sp, P, C, center):
    Gp, K, _ = g3.shape
    lot = sp['lin_w'].reshape(C, C, 16).transpose(2, 1, 0)
    args = [g3]
    if center:
        args.append(nx)
    args += _chain_args(sp['mlp']) + _chain_args(sp['wn']) + _chain_args(sp['dn'])
    args += [lot, _r2(sp['lin_b']), _r2(sp['bnl_g']), _r2(sp['bnl_b'])]
    body = _make_grouped_body(Gp, K, P, C, center)
    return pl.pallas_call(
        body,
        out_shape=jax.ShapeDtypeStruct((Gp, C), F32),
    )(*args)


def _ln_elu(x, wt, b, g, bt):
    y = _dotf(x, wt) + b
    m = jnp.mean(y, axis=1, keepdims=True)
    v = jnp.mean((y - m) ** 2, axis=1, keepdims=True)
    y = (y - m) / jnp.sqrt(v + EPS) * g + bt
    return jnp.where(y > 0, y, jnp.exp(jnp.minimum(y, 0.0)) - 1.0)


def _head_body(q_ref, xpc_ref, f1q_w, f1q_b, f1q_g, f1q_t,
               f2q_w, f2q_b, f2q_g, f2q_t, f3q_w, f3q_b, f3q_g, f3q_t,
               w1a, w1b, f1_b, f1_g, f1_t, f2_w, f2_b, f2_g, f2_t,
               w3, b3, out_ref):
    q = q_ref[0]
    xq = _ln_elu(q, f1q_w[...], f1q_b[...], f1q_g[...], f1q_t[...])
    xq = _ln_elu(xq, f2q_w[...], f2q_b[...], f2q_g[...], f2q_t[...])
    xq = _ln_elu(xq, f3q_w[...], f3q_b[...], f3q_g[...], f3q_t[...])
    pc_part = _dotf(xpc_ref[0], w1a[...])
    h = _dotf(xq, w1b[...]) + pc_part + f1_b[...]
    m = jnp.mean(h, axis=1, keepdims=True)
    v = jnp.mean((h - m) ** 2, axis=1, keepdims=True)
    h = (h - m) / jnp.sqrt(v + EPS) * f1_g[...] + f1_t[...]
    h = jnp.where(h > 0, h, jnp.exp(jnp.minimum(h, 0.0)) - 1.0)
    h = _ln_elu(h, f2_w[...], f2_b[...], f2_g[...], f2_t[...])
    out_ref[0] = _dotf(h, w3[...]) + b3[...]


def _head(query, x_pc, params):
    B, NQ, _ = query.shape
    p1q, p2q, p3q = params['fc1q'], params['fc2q'], params['fc3q']
    p1, p2, p3 = params['fc1'], params['fc2'], params['fc3']
    w1t = p1['w'].T
    args = [query, x_pc.reshape(B, 1, 256),
            p1q['w'].T, _r2(p1q['b']), _r2(p1q['g']), _r2(p1q['beta']),
            p2q['w'].T, _r2(p2q['b']), _r2(p2q['g']), _r2(p2q['beta']),
            p3q['w'].T, _r2(p3q['b']), _r2(p3q['g']), _r2(p3q['beta']),
            w1t[:256], w1t[256:], _r2(p1['b']), _r2(p1['g']), _r2(p1['beta']),
            p2['w'].T, _r2(p2['b']), _r2(p2['g']), _r2(p2['beta']),
            p3['w'].T, _r2(p3['b'])]
    full = lambda shp: pl.BlockSpec(shp, lambda b: tuple(0 for _ in shp))
    in_specs = [pl.BlockSpec((1, NQ, 3), lambda b: (b, 0, 0)),
                pl.BlockSpec((1, 1, 256), lambda b: (b, 0, 0))]
    in_specs += [full(a.shape) for a in args[2:]]
    out = pl.pallas_call(
        _head_body,
        grid=(B,),
        in_specs=in_specs,
        out_specs=pl.BlockSpec((1, NQ, 1), lambda b: (b, 0, 0)),
        out_shape=jax.ShapeDtypeStruct((B, NQ, 1), F32),
    )(*args)
    return out.reshape(B * NQ, 1)


def _chans_conv(chans, chain_refs):
    for (wt_ref, b_ref, g_ref, bt_ref) in chain_refs:
        wt = wt_ref[...]
        cin, cout = wt.shape
        if cin > 1:
            wt = wt.astype(jnp.bfloat16).astype(F32)
            cb = [c.astype(jnp.bfloat16).astype(F32) for c in chans]
        else:
            cb = chans
        out = []
        for co in range(cout):
            y = cb[0] * wt[0, co] + b_ref[0, co]
            for ci in range(1, cin):
                y = y + cb[ci] * wt[ci, co]
            m = jnp.mean(y)
            v = jnp.mean((y - m) ** 2)
            y = (y - m) / jnp.sqrt(v + EPS) * g_ref[0, co] + bt_ref[0, co]
            out.append(jnp.maximum(y, 0.0))
        chans = out
    return chans


def _wndn_body(*refs):
    (gx0, gx1, gx2, nx0, nx1, nx2, gi) = refs[:7]
    i = 7
    chain_refs = []
    for _ in range(6):
        chain_refs.append(tuple(refs[i:i + 4]))
        i += 4
    w16_ref, ds_ref = refs[i], refs[i + 1]
    wn, dn = chain_refs[:3], chain_refs[3:]
    xyzn = [gx0[...] - nx0[...], gx1[...] - nx1[...], gx2[...] - nx2[...]]
    wch = _chans_conv(xyzn, wn)
    for o in range(16):
        w16_ref[o] = wch[o]
    invd = gi[...]
    mx = jnp.max(invd, axis=0, keepdims=True)
    dch = _chans_conv([invd / mx], dn)
    ds_ref[...] = dch[0]


def _wndn(gx, nx, gi, sp):
    K, Gp, _ = gx.shape
    args = [gx[:, :, 0], gx[:, :, 1], gx[:, :, 2],
            nx[:, 0].reshape(1, Gp), nx[:, 1].reshape(1, Gp),
            nx[:, 2].reshape(1, Gp), gi]
    args += _chain_args(sp['wn']) + _chain_args(sp['dn'])
    return pl.pallas_call(
        _wndn_body,
        out_shape=[jax.ShapeDtypeStruct((16, K, Gp), F32),
                   jax.ShapeDtypeStruct((K, Gp), F32)],
    )(*args)


def _mlp1_body(P, K, g_ref, nx_ref, w_ref, b_ref, y_ref, s1_ref, s2_ref):
    k = pl.program_id(0)
    gk = g_ref[0]
    np8 = jnp.concatenate([gk[:, 0:3] - nx_ref[...], gk[:, 3:3 + P]], axis=1)
    y = _dotf(np8, w_ref[...]) + b_ref[...]
    y_ref[0] = y
    s1 = jnp.sum(y, axis=0, keepdims=True)
    s2 = jnp.sum(y * y, axis=0, keepdims=True)

    @pl.when(k == 0)
    def _():
        s1_ref[...] = s1
        s2_ref[...] = s2

    @pl.when(k > 0)
    def _():
        s1_ref[...] = s1_ref[...] + s1
        s2_ref[...] = s2_ref[...] + s2


def _mlp1(g3km, nx, sp, P, C):
    K, Gp, Wt = g3km.shape
    body = functools.partial(_mlp1_body, P, K)
    L = sp['mlp'][0]
    return pl.pallas_call(
        body,
        grid=(K,),
        in_specs=[
            pl.BlockSpec((1, Gp, Wt), lambda k: (k, 0, 0)),
            pl.BlockSpec((Gp, 3), lambda k: (0, 0)),
            pl.BlockSpec((3 + P, C), lambda k: (0, 0)),
            pl.BlockSpec((1, C), lambda k: (0, 0)),
        ],
        out_specs=[
            pl.BlockSpec((1, Gp, C), lambda k: (k, 0, 0)),
            pl.BlockSpec((1, C), lambda k: (0, 0)),
            pl.BlockSpec((1, C), lambda k: (0, 0)),
        ],
        out_shape=[
            jax.ShapeDtypeStruct((K, Gp, C), F32),
            jax.ShapeDtypeStruct((1, C), F32),
            jax.ShapeDtypeStruct((1, C), F32),
        ],
    )(g3km, nx, L['w'].T, _r2(L['b']))


def _ctr_body(K, C, R, y_ref, s1_ref, s2_ref, mg_ref, mb_ref, ds_ref, w16_ref,
              lot_ref, linb_ref, bg_ref, bb_ref, out_ref, acc_ref):
    k = pl.program_id(0)
    m = s1_ref[...] * (1.0 / R)
    v = s2_ref[...] * (1.0 / R) - m * m
    x = (y_ref[0] - m) / jnp.sqrt(v + EPS) * mg_ref[...] + mb_ref[...]
    x = jnp.maximum(x, 0.0)
    x = x * ds_ref[0]
    x = x.astype(jnp.bfloat16).astype(F32)
    wk = w16_ref[0].astype(jnp.bfloat16).astype(F32)
    f = jnp.concatenate([x * wk[:, o:o + 1] for o in range(16)], axis=1)

    @pl.when(k == 0)
    def _():
        acc_ref[...] = f

    @pl.when(k > 0)
    def _():
        acc_ref[...] = acc_ref[...] + f

    @pl.when(k == K - 1)
    def _():
        y = _dotf(acc_ref[...], lot_ref[...]) + linb_ref[...]
        mm = jnp.mean(y, axis=0, keepdims=True)
        vv = jnp.mean((y - mm) ** 2, axis=0, keepdims=True)
        y = (y - mm) / jnp.sqrt(vv + EPS) * bg_ref[...] + bb_ref[...]
        out_ref[...] = jnp.maximum(y, 0.0)


def _contract(y3, s1, s2, ds3, w16, sp, C):
    K, Gp, _ = y3.shape
    L = sp['mlp'][0]
    linf = sp['lin_w'].reshape(C, C, 16).transpose(2, 1, 0).reshape(16 * C, C)
    body = functools.partial(_ctr_body, K, C, K * Gp)
    full2 = lambda a: pl.BlockSpec(a.shape, lambda k: (0,) * a.ndim)
    args = [y3, s1, s2, _r2(L['g']), _r2(L['beta']), ds3, w16,
            linf, _r2(sp['lin_b']), _r2(sp['bnl_g']), _r2(sp['bnl_b'])]
    return pl.pallas_call(
        body,
        grid=(K,),
        in_specs=[pl.BlockSpec((1, Gp, C), lambda k: (k, 0, 0)),
                  full2(s1), full2(s2), full2(args[3]), full2(args[4]),
                  pl.BlockSpec((1, Gp, 1), lambda k: (k, 0, 0)),
                  pl.BlockSpec((1, Gp, 16), lambda k: (k, 0, 0)),
                  full2(linf), full2(args[8]), full2(args[9]), full2(args[10])],
        out_specs=pl.BlockSpec((Gp, C), lambda k: (0, 0)),
        out_shape=jax.ShapeDtypeStruct((Gp, C), F32),
        scratch_shapes=[pltpu.VMEM((Gp, 16 * C), F32)],
    )(*args)


def _level(xyz_bn3, pts_bnc, params_sa, S, K, bw, C):
    B, N, _ = xyz_bn3.shape
    P = pts_bnc.shape[2]
    Gp = B * S
    fps_idx = _fps(xyz_bn3.transpose(2, 0, 1), S)
    invd, nxyz, knn = _geo(xyz_bn3, fps_idx.reshape(B, S, 1), K, bw)
    Wt = 4 + P + (-(4 + P)) % 16
    table = jnp.concatenate(
        [xyz_bn3, pts_bnc, invd.reshape(B, N, 1),
         jnp.zeros((B, N, Wt - 4 - P), F32)], axis=2).reshape(B * N, Wt)
    flat_idx = (knn + (jnp.arange(B, dtype=jnp.int32) * N)[:, None, None])
    flat_idx = flat_idx.transpose(2, 0, 1).reshape(-1)
    g = _sc_gather(table, flat_idx)
    g3km = g.reshape(K, Gp, Wt)
    nx = nxyz.reshape(Gp, 3)
    w16, ds = _wndn(g3km[:, :, 0:3], nx, g3km[:, :, 3 + P], params_sa)
    y3, s1, s2 = _mlp1(g3km, nx, params_sa, P, C)
    w16t = w16.transpose(1, 2, 0)
    pts = _contract(y3, s1, s2, ds.reshape(K, Gp, 1), w16t, params_sa, C)
    return nxyz, pts.reshape(B, S, C)


def kernel(pc, query, params):
    B = pc.shape[0]
    xyz0 = pc[:, :3, :].transpose(0, 2, 1)
    pts0 = pc.transpose(0, 2, 1)
    l1_xyz, l1_pts = _level(xyz0, pts0, params['sa1'], 512, 32, 0.1, 64)
    l2_xyz, l2_pts = _level(l1_xyz, l1_pts, params['sa2'], 128, 64, 0.2, 128)
    invd3 = _dens(l2_xyz, 0.4)
    t3 = jnp.concatenate([l2_xyz, l2_pts, invd3], axis=2)
    x_pc = _grouped(t3, None, params['sa3'], 128, 256, False)
    return _head(query, x_pc, params)
